# baseline (speedup 1.0000x reference)
import functools

import jax
import jax.numpy as jnp
from jax import lax
from jax.experimental import pallas as pl
from jax.experimental.pallas import tpu as pltpu

_R = 8


def _select_x(k, kp, s, eos_b, row8, tp, feat):
    ii8 = lax.broadcasted_iota(jnp.int32, (8, 1), 0)
    row = jnp.max(jnp.where(ii8 == lax.rem(eos_b, 8), row8, -jnp.inf),
                  axis=0, keepdims=True)
    mask = (tp != 1).astype(jnp.float32)
    j = lax.broadcasted_iota(jnp.int32, (1, s), 1)
    v = jnp.where((j == 0) | (j == eos_b), -1.0, row) * mask

    bits = lax.bitcast_convert_type(v, jnp.int32)
    key = bits ^ ((bits >> 31) & jnp.int32(0x7FFFFFFF))
    ki = key.reshape(s, 1)
    ii = lax.broadcasted_iota(jnp.int32, (s, 1), 0)
    beats = (ki > key) | ((ki == key) & (ii < j))
    ranks = jnp.sum(beats.astype(jnp.int32), axis=0, keepdims=True)

    r_io = lax.broadcasted_iota(jnp.int32, (kp, s), 0)
    p_sel = ((r_io == ranks) & (r_io < k)).astype(jnp.bfloat16)
    return jnp.dot(p_sel, feat.astype(jnp.bfloat16),
                   preferred_element_type=jnp.float32)


def _stage_a_body(k, kp, s, eos_ref, *refs):
    atten_refs = refs[:_R]
    text_ref, feat_ref, w1t_ref, b1_ref, xn_ref, s1_ref, s2_ref = refs[_R:]
    step = pl.program_id(0)

    xs = []
    for r in range(_R):
        b = step * _R + r
        xs.append(_select_x(k, kp, s, eos_ref[b], atten_refs[r][0],
                            text_ref[r], feat_ref[r]))
    x = jnp.concatenate(xs, axis=0)
    nrm = jnp.sqrt(jnp.sum(x * x, axis=1, keepdims=True)) + 1e-8
    xn = (x / nrm).astype(jnp.bfloat16)

    h = jnp.dot(xn, w1t_ref[...], preferred_element_type=jnp.float32)
    h = h + b1_ref[...]
    hb = h.astype(jnp.bfloat16)

    valid = (lax.broadcasted_iota(jnp.int32, (_R, kp, 1), 1) < k).astype(
        jnp.float32).reshape(_R * kp, 1)
    hm = hb.astype(jnp.float32) * valid
    s1 = jnp.sum(hm, axis=0, keepdims=True)
    s2 = jnp.sum(hm * hm, axis=0, keepdims=True)

    @pl.when(step == 0)
    def _():
        s1_ref[...] = s1
        s2_ref[...] = s2

    @pl.when(step != 0)
    def _():
        s1_ref[...] += s1
        s2_ref[...] += s2

    xn_ref[...] = xn.reshape(_R, kp, -1)


def _stage_b_body(k, kp, s, n_rows, text_ref, xn_ref, s1_ref, s2_ref,
                  w1t_ref, b1_ref, gamma_ref, beta_ref, w2t_ref, b2_ref,
                  wlt_ref, blin_ref, out_ref):
    tp = text_ref[...]
    lengths = jnp.sum((tp != 1).astype(jnp.int32), axis=2, keepdims=True)
    pl_b = jnp.minimum(jnp.maximum(lengths - 2, 1), k)

    mean = s1_ref[...] * (1.0 / n_rows)
    var = s2_ref[...] * (1.0 / n_rows) - mean * mean
    scale = gamma_ref[...] / jnp.sqrt(var + 1e-5)

    xn = xn_ref[...].reshape(_R * kp, -1)
    h = jnp.dot(xn, w1t_ref[...], preferred_element_type=jnp.float32)
    h = (h + b1_ref[...]).astype(jnp.bfloat16).astype(jnp.float32)
    hn = (h - mean) * scale + beta_ref[...]
    hr = jnp.maximum(hn, 0.0).astype(jnp.bfloat16)

    out = jnp.dot(hr, w2t_ref[...], preferred_element_type=jnp.float32)
    out += jnp.dot(xn, wlt_ref[...], preferred_element_type=jnp.float32)
    out += b2_ref[...] + blin_ref[...]

    out3 = out.reshape(_R, kp, -1)
    r_io = lax.broadcasted_iota(jnp.int32, (_R, kp, 1), 1)
    keep = r_io < pl_b
    out_ref[...] = jnp.max(jnp.where(keep, out3, -1e30), axis=1,
                           keepdims=True)


def kernel(features, text, atten, W_lin, b_lin, W1, b1, gamma1, beta1, W2, b2):
    bs, s, d_in = features.shape
    d_emb, h_dim = W2.shape
    k = max(int((atten.shape[1] - 2) * 0.3), 1)
    kp = (k + 7) // 8 * 8
    n_rows = float(bs * k)
    n_steps = bs // _R

    text_p = jnp.concatenate(
        [jnp.zeros((bs, 1), jnp.int32), text], axis=1)
    eos = jnp.clip(jnp.sum((text_p != 1).astype(jnp.int32), axis=1) - 1, 0)
    text3 = text_p.reshape(bs, 1, s)
    w1t = W1.T.astype(jnp.bfloat16)
    w2t = W2.T.astype(jnp.bfloat16)
    wlt = W_lin.T.astype(jnp.bfloat16)

    def atten_spec(r):
        return pl.BlockSpec(
            (1, 8, s),
            lambda b, e, r=r: (b * _R + r, e[b * _R + r] // 8, 0))

    grid_a = pltpu.PrefetchScalarGridSpec(
        num_scalar_prefetch=1,
        grid=(n_steps,),
        in_specs=[atten_spec(r) for r in range(_R)] + [
            pl.BlockSpec((_R, 1, s), lambda b, e: (b, 0, 0)),
            pl.BlockSpec((_R, s, d_in), lambda b, e: (b, 0, 0)),
            pl.BlockSpec((d_in, h_dim), lambda b, e: (0, 0)),
            pl.BlockSpec((1, h_dim), lambda b, e: (0, 0)),
        ],
        out_specs=[
            pl.BlockSpec((_R, kp, d_in), lambda b, e: (b, 0, 0)),
            pl.BlockSpec((1, h_dim), lambda b, e: (0, 0)),
            pl.BlockSpec((1, h_dim), lambda b, e: (0, 0)),
        ],
    )
    xn, s1, s2 = pl.pallas_call(
        functools.partial(_stage_a_body, k, kp, s),
        grid_spec=grid_a,
        out_shape=[
            jax.ShapeDtypeStruct((bs, kp, d_in), jnp.bfloat16),
            jax.ShapeDtypeStruct((1, h_dim), jnp.float32),
            jax.ShapeDtypeStruct((1, h_dim), jnp.float32),
        ],
    )(eos, *([atten] * _R), text3, features, w1t, b1.reshape(1, h_dim))

    pooled = pl.pallas_call(
        functools.partial(_stage_b_body, k, kp, s, n_rows),
        grid=(n_steps,),
        in_specs=[
            pl.BlockSpec((_R, 1, s), lambda b: (b, 0, 0)),
            pl.BlockSpec((_R, kp, d_in), lambda b: (b, 0, 0)),
            pl.BlockSpec((1, h_dim), lambda b: (0, 0)),
            pl.BlockSpec((1, h_dim), lambda b: (0, 0)),
            pl.BlockSpec((d_in, h_dim), lambda b: (0, 0)),
            pl.BlockSpec((1, h_dim), lambda b: (0, 0)),
            pl.BlockSpec((1, h_dim), lambda b: (0, 0)),
            pl.BlockSpec((1, h_dim), lambda b: (0, 0)),
            pl.BlockSpec((h_dim, d_emb), lambda b: (0, 0)),
            pl.BlockSpec((1, d_emb), lambda b: (0, 0)),
            pl.BlockSpec((d_in, d_emb), lambda b: (0, 0)),
            pl.BlockSpec((1, d_emb), lambda b: (0, 0)),
        ],
        out_specs=pl.BlockSpec((_R, 1, d_emb), lambda b: (b, 0, 0)),
        out_shape=jax.ShapeDtypeStruct((bs, 1, d_emb), jnp.float32),
    )(text3, xn, s1, s2, w1t, b1.reshape(1, h_dim),
      gamma1.reshape(1, h_dim), beta1.reshape(1, h_dim),
      w2t, b2.reshape(1, d_emb), wlt, b_lin.reshape(1, d_emb))

    return pooled.reshape(bs, d_emb)


# speedup vs baseline: 1.0965x; 1.0965x over previous
import functools

import jax
import jax.numpy as jnp
from jax import lax
from jax.experimental import pallas as pl
from jax.experimental.pallas import tpu as pltpu

_R = 4


def _select_x(k, kp, s, eos_b, row8, tp, feat):
    ii8 = lax.broadcasted_iota(jnp.int32, (8, 1), 0)
    row = jnp.max(jnp.where(ii8 == lax.rem(eos_b, 8), row8, -jnp.inf),
                  axis=0, keepdims=True)
    mask = (tp != 1).astype(jnp.float32)
    j = lax.broadcasted_iota(jnp.int32, (1, s), 1)
    v = jnp.where((j == 0) | (j == eos_b), -1.0, row) * mask

    bits = lax.bitcast_convert_type(v, jnp.int32)
    key = bits ^ ((bits >> 31) & jnp.int32(0x7FFFFFFF))
    ki = key.reshape(s, 1)
    ii = lax.broadcasted_iota(jnp.int32, (s, 1), 0)
    beats = (ki > key) | ((ki == key) & (ii < j))
    ranks = jnp.sum(beats.astype(jnp.int32), axis=0, keepdims=True)

    r_io = lax.broadcasted_iota(jnp.int32, (kp, s), 0)
    p_sel = ((r_io == ranks) & (r_io < k)).astype(jnp.bfloat16)
    return jnp.dot(p_sel, feat.astype(jnp.bfloat16),
                   preferred_element_type=jnp.float32)


def _body(k, kp, s, n_rows, eos_ref, *refs):
    atten_refs = refs[:_R]
    (text_ref, feat_ref, w1_ref, w2_ref, wl_ref, b1_ref, gamma_ref, beta_ref,
     b2_ref, blin_ref, out_ref, xn_scr, h_scr, s1_scr, s2_scr, w1b_scr,
     w2b_scr, wlb_scr) = refs[_R:]
    phase = pl.program_id(0)
    step = pl.program_id(1)

    @pl.when((phase == 0) & (step == 0))
    def _():
        w1b_scr[...] = w1_ref[...].astype(jnp.bfloat16)
        w2b_scr[...] = w2_ref[...].astype(jnp.bfloat16)
        wlb_scr[...] = wl_ref[...].astype(jnp.bfloat16)

    @pl.when(phase == 0)
    def _():
        xs = []
        for r in range(_R):
            b = step * _R + r
            xs.append(_select_x(k, kp, s, eos_ref[b], atten_refs[r][0],
                                text_ref[r], feat_ref[r]))
        x = jnp.concatenate(xs, axis=0)
        nrm = jnp.sqrt(jnp.sum(x * x, axis=1, keepdims=True)) + 1e-8
        xn = (x / nrm).astype(jnp.bfloat16)

        h = lax.dot_general(xn, w1b_scr[...], (((1,), (1,)), ((), ())),
                            preferred_element_type=jnp.float32)
        h = h + b1_ref[...]
        hb = h.astype(jnp.bfloat16)

        valid = (lax.broadcasted_iota(jnp.int32, (_R, kp, 1), 1) < k).astype(
            jnp.float32).reshape(_R * kp, 1)
        hm = hb.astype(jnp.float32) * valid
        s1 = jnp.sum(hm, axis=0, keepdims=True)
        s2 = jnp.sum(hm * hm, axis=0, keepdims=True)

        @pl.when(step == 0)
        def _():
            s1_scr[...] = s1
            s2_scr[...] = s2

        @pl.when(step != 0)
        def _():
            s1_scr[...] += s1
            s2_scr[...] += s2

        xn_scr[step] = xn
        h_scr[step] = hb

    @pl.when(phase == 1)
    def _():
        tp = text_ref[...]
        lengths = jnp.sum((tp != 1).astype(jnp.int32), axis=2, keepdims=True)
        pl_b = jnp.minimum(jnp.maximum(lengths - 2, 1), k)

        mean = s1_scr[...] * (1.0 / n_rows)
        var = s2_scr[...] * (1.0 / n_rows) - mean * mean
        scale = gamma_ref[...] / jnp.sqrt(var + 1e-5)

        xn = xn_scr[step]
        h = h_scr[step].astype(jnp.float32)
        hn = (h - mean) * scale + beta_ref[...]
        hr = jnp.maximum(hn, 0.0).astype(jnp.bfloat16)

        out = lax.dot_general(hr, w2b_scr[...], (((1,), (1,)), ((), ())),
                              preferred_element_type=jnp.float32)
        out += lax.dot_general(xn, wlb_scr[...], (((1,), (1,)), ((), ())),
                               preferred_element_type=jnp.float32)
        out += b2_ref[...] + blin_ref[...]

        out3 = out.reshape(_R, kp, -1)
        r_io = lax.broadcasted_iota(jnp.int32, (_R, kp, 1), 1)
        keep = r_io < pl_b
        out_ref[...] = jnp.max(jnp.where(keep, out3, -1e30), axis=1,
                               keepdims=True)


def kernel(features, text, atten, W_lin, b_lin, W1, b1, gamma1, beta1, W2, b2):
    bs, s, d_in = features.shape
    d_emb, h_dim = W2.shape
    k = max(int((atten.shape[1] - 2) * 0.3), 1)
    kp = (k + 7) // 8 * 8
    n_rows = float(bs * k)
    n_steps = bs // _R

    text_p = jnp.concatenate(
        [jnp.zeros((bs, 1), jnp.int32), text], axis=1)
    eos = jnp.clip(jnp.sum((text_p != 1).astype(jnp.int32), axis=1) - 1, 0)
    text3 = text_p.reshape(bs, 1, s)

    def atten_spec(r):
        def imap(p, b, e, r=r):
            bb = (1 - p) * (b * _R + r) + p * (bs - _R + r)
            return (bb, e[bb] // 8, 0)
        return pl.BlockSpec((1, 8, s), imap)

    grid = pltpu.PrefetchScalarGridSpec(
        num_scalar_prefetch=1,
        grid=(2, n_steps),
        in_specs=[atten_spec(r) for r in range(_R)] + [
            pl.BlockSpec((_R, 1, s), lambda p, b, e: (b, 0, 0)),
            pl.BlockSpec((_R, s, d_in),
                         lambda p, b, e: ((1 - p) * b + p * (bs // _R - 1),
                                          0, 0)),
            pl.BlockSpec((h_dim, d_in), lambda p, b, e: (0, 0)),
            pl.BlockSpec((d_emb, h_dim), lambda p, b, e: (0, 0)),
            pl.BlockSpec((d_emb, d_in), lambda p, b, e: (0, 0)),
            pl.BlockSpec((1, h_dim), lambda p, b, e: (0, 0)),
            pl.BlockSpec((1, h_dim), lambda p, b, e: (0, 0)),
            pl.BlockSpec((1, h_dim), lambda p, b, e: (0, 0)),
            pl.BlockSpec((1, d_emb), lambda p, b, e: (0, 0)),
            pl.BlockSpec((1, d_emb), lambda p, b, e: (0, 0)),
        ],
        out_specs=pl.BlockSpec((_R, 1, d_emb), lambda p, b, e: (b, 0, 0)),
        scratch_shapes=[
            pltpu.VMEM((n_steps, _R * kp, d_in), jnp.bfloat16),
            pltpu.VMEM((n_steps, _R * kp, h_dim), jnp.bfloat16),
            pltpu.VMEM((1, h_dim), jnp.float32),
            pltpu.VMEM((1, h_dim), jnp.float32),
            pltpu.VMEM((h_dim, d_in), jnp.bfloat16),
            pltpu.VMEM((d_emb, h_dim), jnp.bfloat16),
            pltpu.VMEM((d_emb, d_in), jnp.bfloat16),
        ],
    )
    pooled = pl.pallas_call(
        functools.partial(_body, k, kp, s, n_rows),
        grid_spec=grid,
        out_shape=jax.ShapeDtypeStruct((bs, 1, d_emb), jnp.float32),
    )(eos, *([atten] * _R), text3, features, W1, W2, W_lin,
      b1.reshape(1, h_dim), gamma1.reshape(1, h_dim), beta1.reshape(1, h_dim),
      b2.reshape(1, d_emb), b_lin.reshape(1, d_emb))

    return pooled.reshape(bs, d_emb)
